# baseline (speedup 1.0000x reference)
import functools

import jax
import jax.numpy as jnp
from jax import lax
from jax.experimental import pallas as pl
from jax.experimental.pallas import tpu as pltpu
from jax.experimental.pallas import tpu_sc as plsc

_NUM_EMB = 1000000
_D = 32
_B = 16384
_H = 200

_TOT = _B * _H
_NC, _NS = 2, 16
_NW = _NC * _NS
_PER_W = _TOT // _NW
_G = 128
_NG = 8
_BLK = _G * _NG
_NBLK = _PER_W // _BLK


@functools.partial(
    pl.kernel,
    mesh=plsc.VectorSubcoreMesh(core_axis_name="c", subcore_axis_name="s"),
    out_type=jax.ShapeDtypeStruct((_TOT, _D), jnp.float32),
    compiler_params=pltpu.CompilerParams(use_tc_tiling_on_sc=False),
    scratch_types=[
        pltpu.VMEM((_BLK,), jnp.int32),
        pltpu.VMEM((_BLK, _D), jnp.float32),
        pltpu.SemaphoreType.DMA,
    ],
)
def _emb_gather(idx_hbm, tab_hbm, out_hbm, idx_v, rows_v, sem):
    wid = lax.axis_index("s") * _NC + lax.axis_index("c")
    base = wid * _PER_W

    def body(g, carry):
        start = pl.multiple_of(base + g * _BLK, 8)
        pltpu.sync_copy(idx_hbm.at[pl.ds(start, _BLK)], idx_v)
        pltpu.async_copy(tab_hbm.at[idx_v], rows_v, sem).wait()
        pltpu.sync_copy(rows_v, out_hbm.at[pl.ds(start, _BLK)])
        return carry

    lax.fori_loop(0, _NBLK, body, 0)


def kernel(token_ids, weight):
    flat = token_ids.reshape(_TOT)
    out = _emb_gather(flat, weight)
    return out.reshape(_B, _H, _D)


# speedup vs baseline: 1.0175x; 1.0175x over previous
import functools

import jax
import jax.numpy as jnp
from jax import lax
from jax.experimental import pallas as pl
from jax.experimental.pallas import tpu as pltpu
from jax.experimental.pallas import tpu_sc as plsc

_NUM_EMB = 1000000
_D = 32
_B = 16384
_H = 200

_TOT = _B * _H
_NC, _NS = 2, 16
_NW = _NC * _NS
_PER_W = _TOT // _NW
_BLK = 512
_NBLK = _PER_W // _BLK
_NBUF = 5
_ROUNDS = _NBLK // _NBUF


@functools.partial(
    pl.kernel,
    mesh=plsc.VectorSubcoreMesh(core_axis_name="c", subcore_axis_name="s"),
    out_type=jax.ShapeDtypeStruct((_TOT, _D), jnp.float32),
    compiler_params=pltpu.CompilerParams(use_tc_tiling_on_sc=False),
    scratch_types=[
        pltpu.VMEM((_NBUF, _BLK), jnp.int32),
        pltpu.VMEM((_NBUF, _BLK, _D), jnp.float32),
    ]
    + [pltpu.SemaphoreType.DMA] * _NBUF,
)
def _emb_gather(idx_hbm, tab_hbm, out_hbm, idx_v, rows_v, *sems):
    wid = lax.axis_index("s") * _NC + lax.axis_index("c")
    base = wid * _PER_W

    def load_and_fire(g, b):
        start = pl.multiple_of(base + g * _BLK, 8)
        pltpu.sync_copy(idx_hbm.at[pl.ds(start, _BLK)], idx_v.at[b])
        pltpu.async_copy(tab_hbm.at[idx_v.at[b]], rows_v.at[b], sems[b])

    def drain_and_store(g_old, b):
        pltpu.make_async_copy(
            tab_hbm.at[idx_v.at[b]], rows_v.at[b], sems[b]
        ).wait()
        start = pl.multiple_of(base + g_old * _BLK, 8)
        pltpu.sync_copy(rows_v.at[b], out_hbm.at[pl.ds(start, _BLK)])

    for b in range(_NBUF):
        load_and_fire(b, b)

    def body(r, carry):
        for b in range(_NBUF):
            g = r * _NBUF + b
            drain_and_store(g - _NBUF, b)
            load_and_fire(g, b)
        return carry

    lax.fori_loop(1, _ROUNDS, body, 0)

    for b in range(_NBUF):
        drain_and_store(_NBLK - _NBUF + b, b)


def kernel(token_ids, weight):
    flat = token_ids.reshape(_TOT)
    out = _emb_gather(flat, weight)
    return out.reshape(_B, _H, _D)
